# baseline (speedup 1.0000x reference)
import functools

import jax
import jax.numpy as jnp
from jax import lax
from jax.experimental import pallas as pl
from jax.experimental.pallas import tpu as pltpu
from jax.experimental.pallas import tpu_sc as plsc

LANES = 16
TPB = 64
BLKC = 512 * TPB
NBLK_A = 16
VSPLIT = NBLK_A * BLKC


def _sigma(v):
    return (v & -512) + ((v & 127) << 2) + ((v >> 7) & 3)


def _tc_relayout(tabT, blk0, nblk, zero_last):
    nreal = nblk - 1 if zero_last else nblk

    def body(in_ref, o_ref):
        i = pl.program_id(0)

        @pl.when(i < nreal)
        def _():
            for t in range(TPB):
                p = in_ref[:, t * 512:(t + 1) * 512]
                s = jnp.concatenate(
                    [p[:, j * 128:(j + 1) * 128] for j in range(4)], axis=0)
                o_ref[t * 128:(t + 1) * 128, :] = s.T

        if zero_last:
            @pl.when(i == nreal)
            def _():
                o_ref[...] = jnp.zeros((BLKC // 4, 128), jnp.float32)

    return pl.pallas_call(
        body,
        grid=(nblk,),
        in_specs=[pl.BlockSpec(
            (32, BLKC), lambda i: (0, jnp.where(i < nreal, blk0 + i, 0)))],
        out_specs=pl.BlockSpec((BLKC // 4, 128), lambda i: (i, 0)),
        out_shape=jax.ShapeDtypeStruct((nblk * BLKC // 4, 128), jnp.float32),
    )(tabT)


def _sc_pool(x, table, half, partial=None, denom=None):
    B, SEQ = x.shape
    _, D = table.shape
    NC, NS = 2, 16
    NW = NC * NS
    RPW = B // NW
    C0 = 128
    C1 = SEQ - C0
    NCHUNK = SEQ // LANES
    REM = SEQ - NCHUNK * LANES
    ZSLOT = (table.shape[0] // BLKC - 1) * BLKC if half else 0

    mesh = plsc.VectorSubcoreMesh(core_axis_name="c", subcore_axis_name="s")

    if half == 0:
        out_type = [jax.ShapeDtypeStruct((B, D), jnp.float32),
                    jax.ShapeDtypeStruct((B, LANES), jnp.float32)]
    else:
        out_type = jax.ShapeDtypeStruct((B, D), jnp.float32)

    scratch = [
        pltpu.VMEM((RPW, SEQ), jnp.int32),
        pltpu.VMEM((SEQ, D), jnp.float32),
        pltpu.VMEM((SEQ, D), jnp.float32),
        pltpu.VMEM((RPW, D), jnp.float32),
        pltpu.VMEM((RPW, LANES), jnp.float32),
        pltpu.SemaphoreType.DMA,
        pltpu.SemaphoreType.DMA,
    ]
    if half == 1:
        scratch.insert(4, pltpu.VMEM((RPW, D), jnp.float32))

    @functools.partial(
        pl.kernel,
        out_type=out_type,
        mesh=mesh,
        scratch_types=scratch,
        compiler_params=pltpu.CompilerParams(
            use_tc_tiling_on_sc=False, needs_layout_passes=False),
    )
    def k(*refs):
        if half == 0:
            (x_hbm, tab_hbm, sum_hbm, den_hbm,
             xv, rows0, rows1, accv, denv, sem0, sem1) = refs
            partv = part_hbm = None
        else:
            (x_hbm, tab_hbm, part_hbm, den_hbm, mean_hbm,
             xv, rows0, rows1, accv, partv, denv, sem0, sem1) = refs
        wid = lax.axis_index("s") * NC + lax.axis_index("c")
        base = wid * RPW
        pltpu.sync_copy(x_hbm.at[pl.ds(base, RPW)], xv)
        if half == 1:
            pltpu.sync_copy(part_hbm.at[pl.ds(base, RPW)], partv)
            pltpu.sync_copy(den_hbm.at[pl.ds(base, RPW)], denv)

        lane = lax.iota(jnp.int32, LANES)

        def remap(v):
            if half == 0:
                return jnp.where(v < VSPLIT, _sigma(v),
                                 jnp.zeros((LANES,), jnp.int32))
            return jnp.where(v >= VSPLIT, _sigma(v - VSPLIT),
                             jnp.full((LANES,), ZSLOT, jnp.int32))

        def prep_row(i, carry):
            cnt = jnp.zeros((LANES,), jnp.int32)
            for c in range(NCHUNK):
                v = xv[i, pl.ds(c * LANES, LANES)]
                if half == 0:
                    cnt = cnt + plsc.all_reduce_population_count(v != 0)
                xv[i, pl.ds(c * LANES, LANES)] = remap(v)
            if REM:
                v = xv[i, pl.ds(SEQ - LANES, LANES)]
                keep = lane < (LANES - REM)
                if half == 0:
                    cnt = cnt + plsc.all_reduce_population_count(
                        (~keep) & (v != 0))
                xv[i, pl.ds(SEQ - LANES, LANES)] = jnp.where(
                    keep, v, remap(v))
            if half == 0:
                denv[i, pl.ds(0, LANES)] = jnp.maximum(
                    cnt.astype(jnp.float32), jnp.ones((LANES,), jnp.float32))
            return carry

        lax.fori_loop(0, RPW, prep_row, 0)

        def issue(i, rows, sem):
            pltpu.async_copy(
                tab_hbm.at[xv.at[i, pl.ds(0, C0)]], rows.at[pl.ds(0, C0)], sem)
            pltpu.async_copy(
                tab_hbm.at[xv.at[i, pl.ds(C0, C1)]], rows.at[pl.ds(C0, C1)], sem)

        def drain(rows, sem):
            pltpu.make_async_copy(tab_hbm.at[pl.ds(0, SEQ)], rows, sem).wait()

        def compute(i, rows):
            H = SEQ // 2

            def body(j, carry):
                a0, a1, b0, b1 = carry
                a0 = a0 + rows[j, pl.ds(0, LANES)]
                a1 = a1 + rows[j, pl.ds(LANES, LANES)]
                b0 = b0 + rows[j + H, pl.ds(0, LANES)]
                b1 = b1 + rows[j + H, pl.ds(LANES, LANES)]
                return a0, a1, b0, b1

            z = jnp.zeros((LANES,), jnp.float32)
            a0, a1, b0, b1 = lax.fori_loop(0, H, body, (z, z, z, z), unroll=10)
            a0 = a0 + b0
            a1 = a1 + b1
            if half == 0:
                accv[i, pl.ds(0, LANES)] = a0
                accv[i, pl.ds(LANES, LANES)] = a1
            else:
                d = denv[i, pl.ds(0, LANES)]
                accv[i, pl.ds(0, LANES)] = (a0 + partv[i, pl.ds(0, LANES)]) / d
                accv[i, pl.ds(LANES, LANES)] = (
                    a1 + partv[i, pl.ds(LANES, LANES)]) / d

        issue(0, rows0, sem0)

        def body2(t, carry):
            i0 = t * 2
            issue(i0 + 1, rows1, sem1)
            drain(rows0, sem0)
            compute(i0, rows0)

            @pl.when(i0 + 2 < RPW)
            def _():
                issue(i0 + 2, rows0, sem0)

            drain(rows1, sem1)
            compute(i0 + 1, rows1)
            return carry

        lax.fori_loop(0, RPW // 2, body2, 0)
        if half == 0:
            pltpu.sync_copy(accv, sum_hbm.at[pl.ds(base, RPW)])
            pltpu.sync_copy(denv, den_hbm.at[pl.ds(base, RPW)])
        else:
            pltpu.sync_copy(accv, mean_hbm.at[pl.ds(base, RPW)])

    if half == 0:
        return k(x, table)
    return k(x, table, partial, denom)


def _mm(mean, W, b2):
    B, D = mean.shape
    C, _ = W.shape
    BLK = 256

    def mmk(m_ref, w_ref, b_ref, o_ref):
        o_ref[...] = lax.dot_general(
            m_ref[...], w_ref[...],
            dimension_numbers=(((1,), (1,)), ((), ())),
            preferred_element_type=jnp.float32,
        ) + b_ref[...]

    return pl.pallas_call(
        mmk,
        grid=(B // BLK,),
        in_specs=[
            pl.BlockSpec((BLK, D), lambda i: (i, 0)),
            pl.BlockSpec((C, D), lambda i: (0, 0)),
            pl.BlockSpec((1, C), lambda i: (0, 0)),
        ],
        out_specs=pl.BlockSpec((BLK, C), lambda i: (i, 0)),
        out_shape=jax.ShapeDtypeStruct((B, C), jnp.float32),
    )(mean, W, b2)


@jax.jit
def kernel(x, table, W, b):
    x = x.astype(jnp.int32)
    V, D = table.shape
    tabT = table.T
    nblk_b = -(-(V - VSPLIT) // BLKC) + 1
    packed_a = _tc_relayout(tabT, 0, NBLK_A, zero_last=False)
    packed_b = _tc_relayout(tabT, NBLK_A, nblk_b, zero_last=True)
    tab_a = packed_a.reshape(packed_a.shape[0] * 4, D)
    tab_b = packed_b.reshape(packed_b.shape[0] * 4, D)
    sums_a, denom = _sc_pool(x, tab_a, half=0)
    mean = _sc_pool(x, tab_b, half=1, partial=sums_a, denom=denom)
    return _mm(mean, W, b.reshape(1, -1))


# speedup vs baseline: 39.2794x; 39.2794x over previous
import functools

import jax
import jax.numpy as jnp
from jax import lax
from jax.experimental import pallas as pl
from jax.experimental.pallas import tpu as pltpu
from jax.experimental.pallas import tpu_sc as plsc

LANES = 16


def _sc_pool(x, table):
    B, SEQ = x.shape
    _, D = table.shape
    NC, NS = 2, 16
    NW = NC * NS
    RPW = B // NW
    C0 = 128
    C1 = SEQ - C0

    mesh = plsc.VectorSubcoreMesh(core_axis_name="c", subcore_axis_name="s")

    @functools.partial(
        pl.kernel,
        out_type=jax.ShapeDtypeStruct((B, D), jnp.float32),
        mesh=mesh,
        scratch_types=[
            pltpu.VMEM((RPW, SEQ), jnp.int32),
            pltpu.VMEM((SEQ, D), jnp.float32),
            pltpu.VMEM((SEQ, D), jnp.float32),
            pltpu.VMEM((RPW, D), jnp.float32),
            pltpu.SemaphoreType.DMA,
            pltpu.SemaphoreType.DMA,
        ],
        compiler_params=pltpu.CompilerParams(
            use_tc_tiling_on_sc=False, needs_layout_passes=False),
    )
    def k(x_hbm, tab_hbm, mean_hbm, xv, rows0, rows1, meanv, sem0, sem1):
        wid = lax.axis_index("s") * NC + lax.axis_index("c")
        base = wid * RPW
        pltpu.sync_copy(x_hbm.at[pl.ds(base, RPW)], xv)

        NCHUNK = SEQ // LANES
        lane = lax.iota(jnp.int32, LANES)

        def sigma(v):
            return (v & -512) + ((v & 127) << 2) + ((v >> 7) & 3)

        def remap_row(i, carry):
            for c in range(NCHUNK):
                v = xv[i, pl.ds(c * LANES, LANES)]
                xv[i, pl.ds(c * LANES, LANES)] = sigma(v)
            if SEQ % LANES:
                v = xv[i, pl.ds(SEQ - LANES, LANES)]
                keep = lane < (LANES - SEQ % LANES)
                xv[i, pl.ds(SEQ - LANES, LANES)] = jnp.where(
                    keep, v, sigma(v))
            return carry

        lax.fori_loop(0, RPW, remap_row, 0)

        def issue(i, rows, sem):
            pltpu.async_copy(
                tab_hbm.at[xv.at[i, pl.ds(0, C0)]], rows.at[pl.ds(0, C0)], sem)
            pltpu.async_copy(
                tab_hbm.at[xv.at[i, pl.ds(C0, C1)]], rows.at[pl.ds(C0, C1)], sem)

        def drain(rows, sem):
            pltpu.make_async_copy(tab_hbm.at[pl.ds(0, SEQ)], rows, sem).wait()

        def compute(i, rows):
            H = SEQ // 2

            def body(j, carry):
                a0, a1, b0, b1 = carry
                a0 = a0 + rows[j, pl.ds(0, LANES)]
                a1 = a1 + rows[j, pl.ds(LANES, LANES)]
                b0 = b0 + rows[j + H, pl.ds(0, LANES)]
                b1 = b1 + rows[j + H, pl.ds(LANES, LANES)]
                return a0, a1, b0, b1

            z = jnp.zeros((LANES,), jnp.float32)
            a0, a1, b0, b1 = lax.fori_loop(0, H, body, (z, z, z, z), unroll=10)
            a0 = a0 + b0
            a1 = a1 + b1

            cnt = jnp.zeros((LANES,), jnp.int32)
            for kk in range(SEQ // LANES):
                chunk = xv[i, pl.ds(kk * LANES, LANES)]
                cnt = cnt + plsc.all_reduce_population_count(chunk != 0)
            rem = SEQ - (SEQ // LANES) * LANES
            if rem:
                lane = lax.iota(jnp.int32, LANES)
                last = xv[i, pl.ds(SEQ - LANES, LANES)]
                cnt = cnt + plsc.all_reduce_population_count(
                    (lane >= LANES - rem) & (last != 0))
            denom = jnp.maximum(cnt.astype(jnp.float32),
                                jnp.ones((LANES,), jnp.float32))
            meanv[i, pl.ds(0, LANES)] = a0 / denom
            meanv[i, pl.ds(LANES, LANES)] = a1 / denom

        issue(0, rows0, sem0)

        def body2(t, carry):
            i0 = t * 2
            issue(i0 + 1, rows1, sem1)
            drain(rows0, sem0)
            compute(i0, rows0)

            @pl.when(i0 + 2 < RPW)
            def _():
                issue(i0 + 2, rows0, sem0)

            drain(rows1, sem1)
            compute(i0 + 1, rows1)
            return carry

        lax.fori_loop(0, RPW // 2, body2, 0)
        pltpu.sync_copy(meanv, mean_hbm.at[pl.ds(base, RPW)])

    return k(x, table)


def _tc_relayout(tabT):
    _, V = tabT.shape
    TPB = 128
    BLKC = 512 * TPB
    G = -(-V // BLKC)

    def body(in_ref, o_ref):
        for t in range(TPB):
            p = in_ref[:, t * 512:(t + 1) * 512]
            s = jnp.concatenate(
                [p[:, j * 128:(j + 1) * 128] for j in range(4)], axis=0)
            o_ref[t * 128:(t + 1) * 128, :] = s.T

    return pl.pallas_call(
        body,
        grid=(G,),
        in_specs=[pl.BlockSpec((32, BLKC), lambda i: (0, i))],
        out_specs=pl.BlockSpec((BLKC // 4, 128), lambda i: (i, 0)),
        out_shape=jax.ShapeDtypeStruct((G * BLKC // 4, 128), jnp.float32),
    )(tabT)


def _mm(mean, W, b2):
    B, D = mean.shape
    C, _ = W.shape
    BLK = 256

    def mmk(m_ref, w_ref, b_ref, o_ref):
        o_ref[...] = lax.dot_general(
            m_ref[...], w_ref[...],
            dimension_numbers=(((1,), (1,)), ((), ())),
            preferred_element_type=jnp.float32,
        ) + b_ref[...]

    return pl.pallas_call(
        mmk,
        grid=(B // BLK,),
        in_specs=[
            pl.BlockSpec((BLK, D), lambda i: (i, 0)),
            pl.BlockSpec((C, D), lambda i: (0, 0)),
            pl.BlockSpec((1, C), lambda i: (0, 0)),
        ],
        out_specs=pl.BlockSpec((BLK, C), lambda i: (i, 0)),
        out_shape=jax.ShapeDtypeStruct((B, C), jnp.float32),
    )(mean, W, b2)


@jax.jit
def kernel(x, table, W, b):
    x = x.astype(jnp.int32)
    _, D = table.shape
    packed = _tc_relayout(table.T)
    tab_lin = packed.reshape(packed.shape[0] * 4, D)
    mean = _sc_pool(x, tab_lin)
    return _mm(mean, W, b.reshape(1, -1))


# speedup vs baseline: 39.3094x; 1.0008x over previous
import functools

import jax
import jax.numpy as jnp
from jax import lax
from jax.experimental import pallas as pl
from jax.experimental.pallas import tpu as pltpu
from jax.experimental.pallas import tpu_sc as plsc

LANES = 16


def _sc_pool(x, table):
    B, SEQ = x.shape
    _, D = table.shape
    NC, NS = 2, 16
    NW = NC * NS
    RPW = B // NW
    C0 = 128
    C1 = SEQ - C0

    mesh = plsc.VectorSubcoreMesh(core_axis_name="c", subcore_axis_name="s")

    @functools.partial(
        pl.kernel,
        out_type=jax.ShapeDtypeStruct((B, D), jnp.float32),
        mesh=mesh,
        scratch_types=[
            pltpu.VMEM((RPW, SEQ), jnp.int32),
            pltpu.VMEM((SEQ, D), jnp.float32),
            pltpu.VMEM((SEQ, D), jnp.float32),
            pltpu.VMEM((RPW, D), jnp.float32),
            pltpu.SemaphoreType.DMA,
            pltpu.SemaphoreType.DMA,
        ],
        compiler_params=pltpu.CompilerParams(
            use_tc_tiling_on_sc=False, needs_layout_passes=False),
    )
    def k(x_hbm, tab_hbm, mean_hbm, xv, rows0, rows1, meanv, sem0, sem1):
        wid = lax.axis_index("s") * NC + lax.axis_index("c")
        base = wid * RPW
        pltpu.sync_copy(x_hbm.at[pl.ds(base, RPW)], xv)

        NCHUNK = SEQ // LANES
        lane = lax.iota(jnp.int32, LANES)

        def sigma(v):
            return (v & -512) + ((v & 127) << 2) + ((v >> 7) & 3)

        def remap_row(i, carry):
            for c in range(NCHUNK):
                v = xv[i, pl.ds(c * LANES, LANES)]
                xv[i, pl.ds(c * LANES, LANES)] = sigma(v)
            if SEQ % LANES:
                v = xv[i, pl.ds(SEQ - LANES, LANES)]
                keep = lane < (LANES - SEQ % LANES)
                xv[i, pl.ds(SEQ - LANES, LANES)] = jnp.where(
                    keep, v, sigma(v))
            return carry

        lax.fori_loop(0, RPW, remap_row, 0)

        def issue(i, rows, sem):
            pltpu.async_copy(
                tab_hbm.at[xv.at[i, pl.ds(0, C0)]], rows.at[pl.ds(0, C0)], sem)
            pltpu.async_copy(
                tab_hbm.at[xv.at[i, pl.ds(C0, C1)]], rows.at[pl.ds(C0, C1)], sem)

        def drain(rows, sem):
            pltpu.make_async_copy(tab_hbm.at[pl.ds(0, SEQ)], rows, sem).wait()

        def compute(i, rows):
            H = SEQ // 2

            def body(j, carry):
                a0, a1, b0, b1 = carry
                a0 = a0 + rows[j, pl.ds(0, LANES)]
                a1 = a1 + rows[j, pl.ds(LANES, LANES)]
                b0 = b0 + rows[j + H, pl.ds(0, LANES)]
                b1 = b1 + rows[j + H, pl.ds(LANES, LANES)]
                return a0, a1, b0, b1

            z = jnp.zeros((LANES,), jnp.float32)
            a0, a1, b0, b1 = lax.fori_loop(0, H, body, (z, z, z, z), unroll=10)
            a0 = a0 + b0
            a1 = a1 + b1

            cnt = jnp.zeros((LANES,), jnp.int32)
            for kk in range(SEQ // LANES):
                chunk = xv[i, pl.ds(kk * LANES, LANES)]
                cnt = cnt + plsc.all_reduce_population_count(chunk != 0)
            rem = SEQ - (SEQ // LANES) * LANES
            if rem:
                lane = lax.iota(jnp.int32, LANES)
                last = xv[i, pl.ds(SEQ - LANES, LANES)]
                cnt = cnt + plsc.all_reduce_population_count(
                    (lane >= LANES - rem) & (last != 0))
            denom = jnp.maximum(cnt.astype(jnp.float32),
                                jnp.ones((LANES,), jnp.float32))
            meanv[i, pl.ds(0, LANES)] = a0 / denom
            meanv[i, pl.ds(LANES, LANES)] = a1 / denom

        issue(0, rows0, sem0)

        def body2(t, carry):
            i0 = t * 2
            issue(i0 + 1, rows1, sem1)
            drain(rows0, sem0)
            compute(i0, rows0)

            @pl.when(i0 + 2 < RPW)
            def _():
                issue(i0 + 2, rows0, sem0)

            drain(rows1, sem1)
            compute(i0 + 1, rows1)
            return carry

        lax.fori_loop(0, RPW // 2, body2, 0)
        pltpu.sync_copy(meanv, mean_hbm.at[pl.ds(base, RPW)])

    return k(x, table)


def _tc_relayout(tabT):
    _, V = tabT.shape
    TPB = 128
    BLKC = 512 * TPB
    G = -(-V // BLKC)

    def body(in_ref, o_ref):
        for t in range(TPB):
            p = in_ref[:, t * 512:(t + 1) * 512]
            s = jnp.concatenate(
                [p[:, j * 128:(j + 1) * 128] for j in range(4)], axis=0)
            o_ref[t * 128:(t + 1) * 128, :] = s.T

    return pl.pallas_call(
        body,
        grid=(G,),
        in_specs=[pl.BlockSpec((32, BLKC), lambda i: (0, i))],
        out_specs=pl.BlockSpec((BLKC // 4, 128), lambda i: (i, 0)),
        out_shape=jax.ShapeDtypeStruct((G * BLKC // 4, 128), jnp.float32),
        compiler_params=pltpu.CompilerParams(
            dimension_semantics=("parallel",)),
    )(tabT)


def _mm(mean, W, b2):
    B, D = mean.shape
    C, _ = W.shape
    BLK = 256

    def mmk(m_ref, w_ref, b_ref, o_ref):
        o_ref[...] = lax.dot_general(
            m_ref[...], w_ref[...],
            dimension_numbers=(((1,), (1,)), ((), ())),
            preferred_element_type=jnp.float32,
        ) + b_ref[...]

    return pl.pallas_call(
        mmk,
        grid=(B // BLK,),
        in_specs=[
            pl.BlockSpec((BLK, D), lambda i: (i, 0)),
            pl.BlockSpec((C, D), lambda i: (0, 0)),
            pl.BlockSpec((1, C), lambda i: (0, 0)),
        ],
        out_specs=pl.BlockSpec((BLK, C), lambda i: (i, 0)),
        out_shape=jax.ShapeDtypeStruct((B, C), jnp.float32),
    )(mean, W, b2)


@jax.jit
def kernel(x, table, W, b):
    x = x.astype(jnp.int32)
    _, D = table.shape
    packed = _tc_relayout(table.T)
    tab_lin = packed.reshape(packed.shape[0] * 4, D)
    mean = _sc_pool(x, tab_lin)
    return _mm(mean, W, b.reshape(1, -1))
